# baseline (speedup 1.0000x reference)
import functools

import jax
import jax.numpy as jnp
from jax import lax
from jax.experimental import pallas as pl
from jax.experimental.pallas import tpu as pltpu
from jax.experimental.pallas import tpu_sc as plsc

_K = 8192
_C = 32
_ROWS = 32768
_BLK = 512
_NBLK = _ROWS // _BLK
_BETA = 0.25

_NC = 2
_NS = 16
_NW = _NC * _NS
_B_PER_W = _ROWS // _NW


_KSPLIT = _K // 2


_RS = 64
_NSUB = _BLK // _RS
_CHUNK = 128
_NCH = _KSPLIT // _CHUNK


def _tc_body(x_ref, sx_ref, w2_ref, sw_ref, iota_ref, idx_ref, dsum_ref):
    xb = x_ref[...]
    w2 = w2_ref[...]
    xw2 = lax.dot_general(xb, w2, (((1,), (1,)), ((), ())),
                          preferred_element_type=jnp.float32)
    lane = iota_ref[...]
    sx = sx_ref[...]
    sw = sw_ref[...]

    def scan_block(fold_sx):
        idx_parts = []
        dtot = jnp.zeros((), jnp.float32)
        for r in range(_NSUB):
            r0 = r * _RS
            sxr = sx[r0:r0 + _RS, :]
            wins = []
            for wdw in range(2):
                run = jnp.full((_RS, _CHUNK), jnp.inf, jnp.float32)
                ci = jnp.zeros((_RS, _CHUNK), jnp.float32)
                for c in range(_NCH):
                    col = wdw * _KSPLIT + c * _CHUNK
                    xwc = xw2[r0:r0 + _RS, col:col + _CHUNK]
                    if fold_sx:
                        dc = sxr - xwc
                    else:
                        dc = (sxr + sw[:, col:col + _CHUNK]) - xwc
                    p = dc < run
                    run = jnp.where(p, dc, run)
                    ci = jnp.where(p, jnp.float32(c), ci)
                v = jnp.min(run, axis=1, keepdims=True)
                key = jnp.where(run == v, ci * float(_CHUNK) + lane, jnp.inf)
                kidx = jnp.min(key, axis=1, keepdims=True)
                wins.append((v, kidx))
            (v1, k1), (v2, k2) = wins
            a = v1.astype(jnp.bfloat16).astype(jnp.float32)
            take2 = v2 < a
            idxf = jnp.where(take2, k2 + float(_KSPLIT), k1)
            vsel = jnp.where(take2, v2, v1)
            idx_parts.append(idxf[:, 0].astype(jnp.int32))
            dtot = dtot + jnp.sum(vsel)
        idx_ref[0, 0, :] = jnp.concatenate(idx_parts)
        dsum_ref[0, 0, :] = jnp.broadcast_to(dtot, (128,))

    fast = jnp.logical_and(jnp.all(sx >= 8.0), jnp.all(sw < 2.0 ** -21))

    @pl.when(fast)
    def _():
        scan_block(True)

    @pl.when(jnp.logical_not(fast))
    def _():
        scan_block(False)


_tc_call = pl.pallas_call(
    _tc_body,
    grid=(_NBLK,),
    in_specs=[
        pl.BlockSpec((_BLK, _C), lambda i: (i, 0)),
        pl.BlockSpec((_BLK, 1), lambda i: (i, 0)),
        pl.BlockSpec((_K, _C), lambda i: (0, 0)),
        pl.BlockSpec((1, _K), lambda i: (0, 0)),
        pl.BlockSpec((1, _CHUNK), lambda i: (0, 0)),
    ],
    out_specs=[
        pl.BlockSpec((1, 1, _BLK), lambda i: (i, 0, 0)),
        pl.BlockSpec((1, 1, 128), lambda i: (i, 0, 0)),
    ],
    out_shape=[
        jax.ShapeDtypeStruct((_NBLK, 1, _BLK), jnp.int32),
        jax.ShapeDtypeStruct((_NBLK, 1, 128), jnp.float32),
    ],
)


@functools.cache
def _sc_gather_call():
    @functools.partial(
        pl.kernel,
        mesh=plsc.VectorSubcoreMesh(core_axis_name="c", subcore_axis_name="s",
                                    num_cores=_NC, num_subcores=_NS),
        out_type=jax.ShapeDtypeStruct((_ROWS, _C), jnp.float32),
        scratch_types=[
            pltpu.VMEM((_B_PER_W,), jnp.int32),
            pltpu.VMEM((_B_PER_W, _C), jnp.float32),
            pltpu.SemaphoreType.DMA,
        ],
        compiler_params=pltpu.CompilerParams(use_tc_tiling_on_sc=False),
    )
    def _sc_gather(table_hbm, idx_hbm, out_hbm, idx_v, rows_v, sem):
        wid = lax.axis_index("s") * _NC + lax.axis_index("c")
        base = wid * _B_PER_W
        pltpu.sync_copy(idx_hbm.at[pl.ds(base, _B_PER_W)], idx_v)
        pltpu.async_copy(table_hbm.at[idx_v], rows_v, sem).wait()
        pltpu.sync_copy(rows_v, out_hbm.at[pl.ds(base, _B_PER_W)])

    return _sc_gather


def kernel(x, W):
    x_flat = jnp.transpose(x, (0, 2, 3, 1)).reshape(-1, _C)
    sx = jnp.sum(x_flat ** 2, axis=1, keepdims=True)
    sw = jnp.sum(W ** 2, axis=1).reshape(1, _K)
    iota_row = lax.iota(jnp.float32, _CHUNK).reshape(1, _CHUNK)
    idx_blocks, dsums = _tc_call(x_flat, sx, 2.0 * W, sw, iota_row)
    indices_flat = idx_blocks.reshape(_ROWS)
    xq_flat = _sc_gather_call()(W, indices_flat)
    xq = jnp.transpose(xq_flat.reshape(8, 64, 64, _C), (0, 3, 1, 2))
    indices = indices_flat.reshape(8, 64, 64)
    loss = (1.0 + _BETA) * (jnp.sum(dsums[:, 0, 0]) / x.size)
    x_q_st = x + lax.stop_gradient(xq - x)
    return (x_q_st, loss, indices)


# speedup vs baseline: 1.4639x; 1.4639x over previous
import functools

import jax
import jax.numpy as jnp
from jax import lax
from jax.experimental import pallas as pl
from jax.experimental.pallas import tpu as pltpu
from jax.experimental.pallas import tpu_sc as plsc

_K = 8192
_C = 32
_ROWS = 32768
_BLK = 512
_NBLK = _ROWS // _BLK
_BETA = 0.25

_NC = 2
_NS = 16
_NW = _NC * _NS
_B_PER_W = _ROWS // _NW


_KSPLIT = _K // 2


_RS = 64
_NSUB = _BLK // _RS
_CHUNK = 128
_NCH = _KSPLIT // _CHUNK


def _tc_body(x_ref, sx_ref, w2_ref, sw_ref, iota_ref, idx_ref, dsum_ref,
             *, fold_sx):
    xb = x_ref[...]
    w2 = w2_ref[...]
    xw2 = lax.dot_general(xb, w2, (((1,), (1,)), ((), ())),
                          preferred_element_type=jnp.float32)
    lane = iota_ref[...]
    sx = sx_ref[...]
    sw = sw_ref[...]

    idx_parts = []
    dtot = jnp.zeros((), jnp.float32)
    for r in range(_NSUB):
        r0 = r * _RS
        sxr = sx[r0:r0 + _RS, :]
        wins = []
        for wdw in range(2):
            run = jnp.full((_RS, _CHUNK), jnp.inf, jnp.float32)
            ci = jnp.zeros((_RS, _CHUNK), jnp.float32)
            for c in range(_NCH):
                col = wdw * _KSPLIT + c * _CHUNK
                xwc = xw2[r0:r0 + _RS, col:col + _CHUNK]
                if fold_sx:
                    dc = sxr - xwc
                else:
                    dc = (sxr + sw[:, col:col + _CHUNK]) - xwc
                p = dc < run
                run = jnp.where(p, dc, run)
                ci = jnp.where(p, jnp.float32(c), ci)
            v = jnp.min(run, axis=1, keepdims=True)
            key = jnp.where(run == v, ci * float(_CHUNK) + lane, jnp.inf)
            kidx = jnp.min(key, axis=1, keepdims=True)
            wins.append((v, kidx))
        (v1, k1), (v2, k2) = wins
        a = v1.astype(jnp.bfloat16).astype(jnp.float32)
        take2 = v2 < a
        idxf = jnp.where(take2, k2 + float(_KSPLIT), k1)
        vsel = jnp.where(take2, v2, v1)
        idx_parts.append(idxf[:, 0].astype(jnp.int32))
        dtot = dtot + jnp.sum(vsel)
    idx_ref[0, 0, :] = jnp.concatenate(idx_parts)
    dsum_ref[0, 0, :] = jnp.broadcast_to(dtot, (128,))


def _make_tc_call(fold_sx):
    return pl.pallas_call(
        functools.partial(_tc_body, fold_sx=fold_sx),
        grid=(_NBLK,),
        in_specs=[
            pl.BlockSpec((_BLK, _C), lambda i: (i, 0)),
            pl.BlockSpec((_BLK, 1), lambda i: (i, 0)),
            pl.BlockSpec((_K, _C), lambda i: (0, 0)),
            pl.BlockSpec((1, _K), lambda i: (0, 0)),
            pl.BlockSpec((1, _CHUNK), lambda i: (0, 0)),
        ],
        out_specs=[
            pl.BlockSpec((1, 1, _BLK), lambda i: (i, 0, 0)),
            pl.BlockSpec((1, 1, 128), lambda i: (i, 0, 0)),
        ],
        out_shape=[
            jax.ShapeDtypeStruct((_NBLK, 1, _BLK), jnp.int32),
            jax.ShapeDtypeStruct((_NBLK, 1, 128), jnp.float32),
        ],
    )


_tc_call_fold = _make_tc_call(True)
_tc_call_exact = _make_tc_call(False)


@functools.cache
def _sc_gather_call():
    @functools.partial(
        pl.kernel,
        mesh=plsc.VectorSubcoreMesh(core_axis_name="c", subcore_axis_name="s",
                                    num_cores=_NC, num_subcores=_NS),
        out_type=jax.ShapeDtypeStruct((_ROWS, _C), jnp.float32),
        scratch_types=[
            pltpu.VMEM((_B_PER_W,), jnp.int32),
            pltpu.VMEM((_B_PER_W, _C), jnp.float32),
            pltpu.SemaphoreType.DMA,
        ],
        compiler_params=pltpu.CompilerParams(use_tc_tiling_on_sc=False),
    )
    def _sc_gather(table_hbm, idx_hbm, out_hbm, idx_v, rows_v, sem):
        wid = lax.axis_index("s") * _NC + lax.axis_index("c")
        base = wid * _B_PER_W
        pltpu.sync_copy(idx_hbm.at[pl.ds(base, _B_PER_W)], idx_v)
        pltpu.async_copy(table_hbm.at[idx_v], rows_v, sem).wait()
        pltpu.sync_copy(rows_v, out_hbm.at[pl.ds(base, _B_PER_W)])

    return _sc_gather


def kernel(x, W):
    x_flat = jnp.transpose(x, (0, 2, 3, 1)).reshape(-1, _C)
    sx = jnp.sum(x_flat ** 2, axis=1, keepdims=True)
    sw = jnp.sum(W ** 2, axis=1).reshape(1, _K)
    iota_row = lax.iota(jnp.float32, _CHUNK).reshape(1, _CHUNK)
    fast = jnp.logical_and(jnp.min(sx) >= 8.0, jnp.max(sw) < 2.0 ** -21)
    idx_blocks, dsums = lax.cond(
        fast,
        lambda ops: _tc_call_fold(*ops),
        lambda ops: _tc_call_exact(*ops),
        (x_flat, sx, 2.0 * W, sw, iota_row),
    )
    indices_flat = idx_blocks.reshape(_ROWS)
    xq_flat = _sc_gather_call()(W, indices_flat)
    xq = jnp.transpose(xq_flat.reshape(8, 64, 64, _C), (0, 3, 1, 2))
    indices = indices_flat.reshape(8, 64, 64)
    loss = (1.0 + _BETA) * (jnp.sum(dsums[:, 0, 0]) / x.size)
    x_q_st = x + lax.stop_gradient(xq - x)
    return (x_q_st, loss, indices)
